# baseline (speedup 1.0000x reference)
import functools

import jax
import jax.numpy as jnp
from jax import lax
from jax.experimental import pallas as pl
from jax.experimental.pallas import tpu as pltpu
from jax.experimental.pallas import tpu_sc as plsc

MAX_POS = 1000000
EMBED = 64
WIDE = 128

B, T = 4096, 200
NC, NS = 2, 16
NW = NC * NS
BW = B // NW
L = 16
NBB = BW // L

_MESH = plsc.VectorSubcoreMesh(core_axis_name="c", subcore_axis_name="s")


@functools.partial(
    pl.kernel,
    mesh=_MESH,
    out_type=jax.ShapeDtypeStruct((T, EMBED, B), jnp.float32),
    compiler_params=pltpu.CompilerParams(
        use_tc_tiling_on_sc=False, needs_layout_passes=False
    ),
    scratch_types=[
        pltpu.VMEM((T, BW), jnp.int32),
        pltpu.VMEM((T, BW), jnp.int32),
        pltpu.VMEM((BW, WIDE), jnp.float32),
        pltpu.VMEM((BW, WIDE), jnp.float32),
        pltpu.VMEM((EMBED, BW), jnp.float32),
        pltpu.VMEM((EMBED, BW), jnp.float32),
        pltpu.SemaphoreType.DMA,
        pltpu.SemaphoreType.DMA,
        pltpu.SemaphoreType.DMA,
        pltpu.SemaphoreType.DMA,
    ],
)
def _gather(wide_hbm, xt_hbm, out_hbm, idx_v, hs_v, buf0, buf1, tb0, tb1,
            s0, s1, o0, o1):
    wid = lax.axis_index("s") * NC + lax.axis_index("c")
    col0 = wid * BW
    bufs = (buf0, buf1)
    tbufs = (tb0, tb1)
    ssem = (s0, s1)
    osem = (o0, o1)

    pltpu.sync_copy(xt_hbm.at[:, pl.ds(col0, BW)], idx_v)

    @pl.loop(0, T)
    def _split(r):
        for c in range(NBB):
            v = idx_v[r, pl.ds(c * L, L)]
            idx_v[r, pl.ds(c * L, L)] = v >> 1
            hs_v[r, pl.ds(c * L, L)] = (v & 1) << 6

    def fire_stream(t, p):
        pltpu.async_copy(wide_hbm.at[idx_v.at[t]], bufs[p], ssem[p])

    def wait_stream(p):
        pltpu.make_async_copy(
            wide_hbm.at[idx_v.at[0]], bufs[p], ssem[p]
        ).wait()

    def fire_store(t, p):
        pltpu.async_copy(
            tbufs[p], out_hbm.at[t, :, pl.ds(col0, BW)], osem[p]
        )

    def wait_store(p):
        pltpu.make_async_copy(
            tbufs[p], out_hbm.at[0, :, pl.ds(col0, BW)], osem[p]
        ).wait()

    def transpose(t, p):
        buf = bufs[p]
        tbuf = tbufs[p]

        @pl.loop(0, EMBED)
        def _col(e):
            for bb in range(NBB):
                hs = hs_v[t, pl.ds(bb * L, L)]
                bidx = lax.iota(jnp.int32, L) + bb * L
                vals = plsc.load_gather(buf, [bidx, hs + e])
                tbuf[e, pl.ds(bb * L, L)] = vals

    fire_stream(0, 0)

    @pl.loop(0, T, step=2)
    def _steady(g):
        for db in range(2):
            t = g + db
            p = db
            np_ = 1 - db
            wait_stream(p)

            @pl.when(t + 1 < T)
            def _():
                fire_stream(t + 1, np_)

            @pl.when(t >= 2)
            def _():
                wait_store(p)

            transpose(t, p)
            fire_store(t, p)

    wait_store(0)
    wait_store(1)


def kernel(x, positional_encoding):
    wide = positional_encoding.reshape(MAX_POS // 2, WIDE)
    out3 = _gather(wide, x.T)
    return jnp.transpose(out3, (2, 0, 1))


# speedup vs baseline: 1.4277x; 1.4277x over previous
import functools

import jax
import jax.numpy as jnp
from jax import lax
from jax.experimental import pallas as pl
from jax.experimental.pallas import tpu as pltpu
from jax.experimental.pallas import tpu_sc as plsc

MAX_POS = 1000000
EMBED = 64
WIDE = 128

B, T = 4096, 200
NC, NS = 2, 16
NW = NC * NS
BW = B // NW
L = 16
NBB = BW // L

_MESH = plsc.VectorSubcoreMesh(core_axis_name="c", subcore_axis_name="s")


@functools.partial(
    pl.kernel,
    mesh=_MESH,
    out_type=jax.ShapeDtypeStruct((T, EMBED, B), jnp.float32),
    compiler_params=pltpu.CompilerParams(
        use_tc_tiling_on_sc=True, needs_layout_passes=False
    ),
    scratch_types=[
        pltpu.VMEM((T, BW), jnp.int32),
        pltpu.VMEM((BW, WIDE), jnp.float32),
        pltpu.VMEM((BW, WIDE), jnp.float32),
        pltpu.VMEM((EMBED, BW), jnp.float32),
        pltpu.VMEM((EMBED, BW), jnp.float32),
        pltpu.SemaphoreType.DMA,
        pltpu.SemaphoreType.DMA,
        pltpu.SemaphoreType.DMA,
        pltpu.SemaphoreType.DMA,
    ],
)
def _gather(wide_hbm, xt_hbm, out_hbm, idx_v, buf0, buf1, tb0, tb1,
            s0, s1, o0, o1):
    wid = lax.axis_index("s") * NC + lax.axis_index("c")
    col0 = wid * BW
    bufs = (buf0, buf1)
    tbufs = (tb0, tb1)
    ssem = (s0, s1)
    osem = (o0, o1)

    pltpu.sync_copy(xt_hbm.at[:, pl.ds(col0, BW)], idx_v)

    def fire_stream(t, p):
        pltpu.async_copy(wide_hbm.at[idx_v.at[t]], bufs[p], ssem[p])

    def wait_stream(p):
        pltpu.make_async_copy(
            wide_hbm.at[idx_v.at[0]], bufs[p], ssem[p]
        ).wait()

    def fire_store(t, p):
        pltpu.async_copy(
            tbufs[p], out_hbm.at[t, :, pl.ds(col0, BW)], osem[p]
        )

    def wait_store(p):
        pltpu.make_async_copy(
            tbufs[p], out_hbm.at[0, :, pl.ds(col0, BW)], osem[p]
        ).wait()

    def transpose(p):
        buf = bufs[p]
        tbuf = tbufs[p]
        bidxs = [lax.iota(jnp.int32, L) + bb * L for bb in range(NBB)]

        def body(e, carry):
            evec = jnp.full((L,), 0, jnp.int32) + e
            for bb in range(NBB):
                vals = plsc.load_gather(buf, [bidxs[bb], evec])
                tbuf[e, pl.ds(bb * L, L)] = vals
            return carry

        lax.fori_loop(0, EMBED, body, 0, unroll=8)

    fire_stream(0, 0)

    @pl.loop(0, T, step=2)
    def _steady(g):
        for db in range(2):
            t = g + db
            p = db
            np_ = 1 - db
            wait_stream(p)

            @pl.when(t + 1 < T)
            def _():
                fire_stream(t + 1, np_)

            @pl.when(t >= 2)
            def _():
                wait_store(p)

            transpose(p)
            fire_store(t, p)

    wait_store(0)
    wait_store(1)


def kernel(x, positional_encoding):
    wide = jnp.concatenate([positional_encoding, positional_encoding], axis=1)
    out3 = _gather(wide, x.T)
    return jnp.transpose(out3, (2, 0, 1))


# speedup vs baseline: 2.2139x; 1.5506x over previous
import functools

import jax
import jax.numpy as jnp
from jax import lax
from jax.experimental import pallas as pl
from jax.experimental.pallas import tpu as pltpu
from jax.experimental.pallas import tpu_sc as plsc

MAX_POS = 1000000
EMBED = 64

B, T = 4096, 200
NC, NS = 2, 16
NW = NC * NS
ROWS_W = B // NW
CR = 2
NCHUNKS = ROWS_W // CR
SPANS = ((0, 104), (104, 96))


def _make_sc_gather():
    mesh = plsc.VectorSubcoreMesh(core_axis_name="c", subcore_axis_name="s")

    @functools.partial(
        pl.kernel,
        mesh=mesh,
        out_type=jax.ShapeDtypeStruct((B, T, EMBED), jnp.float32),
        compiler_params=pltpu.CompilerParams(use_tc_tiling_on_sc=False),
        scratch_types=[
            pltpu.VMEM((ROWS_W, T), jnp.int32),
            pltpu.VMEM((CR, T, EMBED), jnp.float32),
            pltpu.VMEM((CR, T, EMBED), jnp.float32),
            pltpu.SemaphoreType.DMA,
            pltpu.SemaphoreType.DMA,
            pltpu.SemaphoreType.DMA,
        ],
    )
    def k(tbl_hbm, idx_hbm, out_hbm, idx_v, rows0, rows1, g0, g1, osem):
        wid = lax.axis_index("s") * NC + lax.axis_index("c")
        row0 = wid * ROWS_W
        rows = (rows0, rows1)
        gsem = (g0, g1)

        pltpu.sync_copy(idx_hbm.at[pl.ds(row0, ROWS_W)], idx_v)

        def fire_gathers(ci, buf, sem):
            for dr in range(CR):
                for off, ln in SPANS:
                    pltpu.async_copy(
                        tbl_hbm.at[idx_v.at[ci * CR + dr, pl.ds(off, ln)]],
                        buf.at[dr, pl.ds(off, ln)],
                        sem,
                    )

        def wait_gathers(buf, sem):
            for dr in range(CR):
                for off, ln in SPANS:
                    pltpu.make_async_copy(
                        tbl_hbm.at[idx_v.at[dr, pl.ds(off, ln)]],
                        buf.at[dr, pl.ds(off, ln)],
                        sem,
                    ).wait()

        def fire_store(ci, buf):
            pltpu.async_copy(
                buf, out_hbm.at[pl.ds(row0 + ci * CR, CR)], osem
            )

        def wait_store(ci, buf):
            pltpu.make_async_copy(
                buf, out_hbm.at[pl.ds(row0 + ci * CR, CR)], osem
            ).wait()

        fire_gathers(0, rows[0], gsem[0])
        fire_gathers(1, rows[1], gsem[1])
        wait_gathers(rows[0], gsem[0])
        fire_store(0, rows[0])

        @pl.loop(1, NCHUNKS - 1, step=2)
        def _steady(g):
            for db in range(2):
                i = g + db
                b = (1 + db) % 2
                nb = 1 - b
                wait_store(i - 1, rows[nb])
                fire_gathers(i + 1, rows[nb], gsem[nb])
                wait_gathers(rows[b], gsem[b])
                fire_store(i, rows[b])

        wait_store(NCHUNKS - 2, rows[0])
        wait_gathers(rows[1], gsem[1])
        fire_store(NCHUNKS - 1, rows[1])
        wait_store(NCHUNKS - 1, rows[1])

    return k


_sc_gather = _make_sc_gather()


def kernel(x, positional_encoding):
    return _sc_gather(positional_encoding, x)
